# Initial kernel scaffold; baseline (speedup 1.0000x reference)
import jax
import jax.numpy as jnp
from jax.experimental import pallas as pl


def kernel(x, x_s, edge_index, edge_features, fwd_W1, fwd_b1, fwd_W2, fwd_b2, rev_W1, rev_b1, rev_W2, rev_b2, ln_w, ln_b, r_W1, r_b1, r_W2, r_b2, z_W1, z_b1, z_W2, z_b2, c_W1, c_b1, c_W2, c_b2):
    raise NotImplementedError("write your pallas kernel here")


# speedup vs baseline: 1.0635x; 1.0635x over previous
import functools
import math

import jax
import jax.numpy as jnp
from jax.experimental import pallas as pl
from jax.experimental.pallas import tpu as pltpu

N = 10000
E = 160000
H = 128
S = 16
EF = 16
K = 2
W = 2 * H
GW = 3 * H
GIN = 3 * H

_NBLK = 1000
_EBLK = 1000


def _proj_body(h_ref, xs_ref, lnw_ref, lnb_ref, wh1_ref, ws1_ref, wh2_ref,
               ws2_ref, hn_ref, p1_ref, p2_ref):
    h = h_ref[...]
    mu = jnp.mean(h, axis=-1, keepdims=True)
    var = jnp.mean((h - mu) * (h - mu), axis=-1, keepdims=True)
    hn = (h - mu) * jax.lax.rsqrt(var + 1e-5) * lnw_ref[...] + lnb_ref[...]
    hn_ref[...] = hn
    xs = xs_ref[...]
    p1_ref[...] = (jnp.dot(hn, wh1_ref[...], preferred_element_type=jnp.float32)
                   + jnp.dot(xs, ws1_ref[...], preferred_element_type=jnp.float32))
    p2_ref[...] = (jnp.dot(hn, wh2_ref[...], preferred_element_type=jnp.float32)
                   + jnp.dot(xs, ws2_ref[...], preferred_element_type=jnp.float32))


@jax.jit
def _proj(h, xs, lnw, lnb, wh1, ws1, wh2, ws2):
    grid = (N // _NBLK,)
    full = lambda shape: pl.BlockSpec(shape, lambda i: (0, 0))
    return pl.pallas_call(
        _proj_body,
        grid=grid,
        in_specs=[
            pl.BlockSpec((_NBLK, H), lambda i: (i, 0)),
            pl.BlockSpec((_NBLK, S), lambda i: (i, 0)),
            full((1, H)),
            full((1, H)),
            full((H, W)),
            full((S, W)),
            full((H, W)),
            full((S, W)),
        ],
        out_specs=[
            pl.BlockSpec((_NBLK, H), lambda i: (i, 0)),
            pl.BlockSpec((_NBLK, W), lambda i: (i, 0)),
            pl.BlockSpec((_NBLK, W), lambda i: (i, 0)),
        ],
        out_shape=[
            jax.ShapeDtypeStruct((N, H), jnp.float32),
            jax.ShapeDtypeStruct((N, W), jnp.float32),
            jax.ShapeDtypeStruct((N, W), jnp.float32),
        ],
    )(h, xs, lnw.reshape(1, H), lnb.reshape(1, H), wh1, ws1, wh2, ws2)


def _score_body(g1_ref, g2_ref, ef_ref, we_ref, b1_ref, w2_ref, b2_ref,
                out_ref, *, is_fwd, temp):
    pre = (g1_ref[...] + g2_ref[...]
           + jnp.dot(ef_ref[...], we_ref[...], preferred_element_type=jnp.float32)
           + b1_ref[...])
    t = jnp.sum(jnp.maximum(pre, 0.0) * w2_ref[...], axis=-1, keepdims=True) \
        + b2_ref[0, 0]
    if is_fwd:
        t = jnp.where(t >= 0.0, t, 0.01 * t)
        out_ref[...] = jnp.exp(t * (1.0 / temp))
    else:
        out_ref[...] = 1.0 / (1.0 + jnp.exp(-t))


@functools.partial(jax.jit, static_argnames=("is_fwd",))
def _score(g1, g2, ef, we, b1, w2, b2, *, is_fwd):
    grid = (E // _EBLK,)
    full = lambda shape: pl.BlockSpec(shape, lambda i: (0, 0))
    temp = math.sqrt(float(H))
    out = pl.pallas_call(
        functools.partial(_score_body, is_fwd=is_fwd, temp=temp),
        grid=grid,
        in_specs=[
            pl.BlockSpec((_EBLK, W), lambda i: (i, 0)),
            pl.BlockSpec((_EBLK, W), lambda i: (i, 0)),
            pl.BlockSpec((_EBLK, EF), lambda i: (i, 0)),
            full((EF, W)),
            full((1, W)),
            full((1, W)),
            full((1, 1)),
        ],
        out_specs=pl.BlockSpec((_EBLK, 1), lambda i: (i, 0)),
        out_shape=jax.ShapeDtypeStruct((E, 1), jnp.float32),
    )(g1, g2, ef, we, b1.reshape(1, W), w2.reshape(1, W), b2.reshape(1, 1))
    return out[:, 0]


def _gru_body(x_ref, mf_ref, mr_ref, rw1_ref, rb1_ref, rw2_ref, rb2_ref,
              zw1_ref, zb1_ref, zw2_ref, zb2_ref, cw1_ref, cb1_ref, cw2_ref,
              cb2_ref, fin_ref, z_ref, r_ref):
    x = x_ref[...]
    gi = jnp.concatenate([x, mf_ref[...], mr_ref[...]], axis=-1)

    def mlp(inp, w1, b1, w2, b2):
        hh = jnp.maximum(
            jnp.dot(inp, w1[...], preferred_element_type=jnp.float32) + b1[...],
            0.0)
        return jnp.dot(hh, w2[...], preferred_element_type=jnp.float32) + b2[...]

    r = jax.nn.sigmoid(mlp(gi, rw1_ref, rb1_ref, rw2_ref, rb2_ref))
    z = jax.nn.sigmoid(mlp(gi, zw1_ref, zb1_ref, zw2_ref, zb2_ref))
    ci = jnp.concatenate([r * x, mf_ref[...], mr_ref[...]], axis=-1)
    cand = jnp.tanh(mlp(ci, cw1_ref, cb1_ref, cw2_ref, cb2_ref))
    fin_ref[...] = (1.0 - z) * x + z * cand
    z_ref[...] = z
    r_ref[...] = r


@jax.jit
def _gru(x, mf, mr, rw1, rb1, rw2, rb2, zw1, zb1, zw2, zb2, cw1, cb1, cw2,
         cb2):
    grid = (N // _NBLK,)
    full = lambda shape: pl.BlockSpec(shape, lambda i: (0, 0))
    nb = lambda w: pl.BlockSpec((_NBLK, w), lambda i: (i, 0))
    return pl.pallas_call(
        _gru_body,
        grid=grid,
        in_specs=[
            nb(H), nb(H), nb(H),
            full((GIN, GW)), full((1, GW)), full((GW, H)), full((1, H)),
            full((GIN, GW)), full((1, GW)), full((GW, H)), full((1, H)),
            full((GIN, GW)), full((1, GW)), full((GW, H)), full((1, H)),
        ],
        out_specs=[nb(H), nb(H), nb(H)],
        out_shape=[
            jax.ShapeDtypeStruct((N, H), jnp.float32),
            jax.ShapeDtypeStruct((N, H), jnp.float32),
            jax.ShapeDtypeStruct((N, H), jnp.float32),
        ],
    )(x, mf, mr,
      rw1.T, rb1.reshape(1, GW), rw2.T, rb2.reshape(1, H),
      zw1.T, zb1.reshape(1, GW), zw2.T, zb2.reshape(1, H),
      cw1.T, cb1.reshape(1, GW), cw2.T, cb2.reshape(1, H))


def kernel(x, x_s, edge_index, edge_features, fwd_W1, fwd_b1, fwd_W2, fwd_b2,
           rev_W1, rev_b1, rev_W2, rev_b2, ln_w, ln_b, r_W1, r_b1, r_W2, r_b2,
           z_W1, z_b1, z_W2, z_b2, c_W1, c_b1, c_W2, c_b2):
    src, dst = edge_index[0], edge_index[1]

    def run_direction(W1, b1, W2, b2, first, second, seg, is_fwd):
        h = x
        ws = []
        for i in range(K):
            w1 = W1[i]
            hn, P1, P2 = _proj(
                h, x_s, ln_w, ln_b,
                w1[:, :H].T, w1[:, W:W + S].T,
                w1[:, H:W].T, w1[:, W + S:W + 2 * S].T)
            g1 = P1[first]
            g2 = P2[second]
            e = _score(g1, g2, edge_features, w1[:, W + 2 * S:].T, b1[i],
                       W2[i, 0], b2[i], is_fwd=is_fwd)
            if is_fwd:
                ssum = jax.ops.segment_sum(e, seg, num_segments=N)
                wgt = e / (ssum[seg] + 1e-9)
            else:
                wgt = e
            h = h + jax.ops.segment_sum(hn[first] * wgt[:, None], seg,
                                        num_segments=N)
            ws.append(wgt)
        return h, jnp.stack(ws, axis=-1)

    h_fwd, fwd_ws = run_direction(fwd_W1, fwd_b1, fwd_W2, fwd_b2,
                                  src, dst, dst, True)
    h_rev, rev_ws = run_direction(rev_W1, rev_b1, rev_W2, rev_b2,
                                  dst, src, src, False)

    final, z, r = _gru(x, h_fwd - x, h_rev - x,
                       r_W1, r_b1, r_W2, r_b2,
                       z_W1, z_b1, z_W2, z_b2,
                       c_W1, c_b1, c_W2, c_b2)
    return (final, fwd_ws, rev_ws, z, r)
